# Initial kernel scaffold; baseline (speedup 1.0000x reference)
import functools

import jax
import jax.numpy as jnp
from jax import lax
from jax.experimental import pallas as pl
from jax.experimental.pallas import tpu as pltpu
from jax.experimental.pallas import tpu_sc as plsc

N = 50000
Q = 512
KNN = 16
H = 128
NC = 13
L = 16
NP = 50048
TD = 160
SC_CORES = 2
SC_SUBCORES = 16
NW = SC_CORES * SC_SUBCORES
QTILE = 128
INF = jnp.float32(jnp.inf)


def _make_sc_gather(B, D):
    b_per_w = B // NW
    mesh = plsc.VectorSubcoreMesh(core_axis_name="c", subcore_axis_name="s")

    @functools.partial(
        pl.kernel,
        out_type=jax.ShapeDtypeStruct((B, D), jnp.float32),
        mesh=mesh,
        scratch_types=[
            pltpu.VMEM((b_per_w,), jnp.int32),
            pltpu.VMEM((b_per_w, D), jnp.float32),
            pltpu.SemaphoreType.DMA,
        ],
    )
    def gather_kernel(table_hbm, idx_hbm, out_hbm, idx_v, rows_v, sem):
        wid = lax.axis_index("s") * SC_CORES + lax.axis_index("c")
        base = wid * b_per_w
        pltpu.sync_copy(idx_hbm.at[pl.ds(base, b_per_w)], idx_v)
        pltpu.async_copy(table_hbm.at[idx_v], rows_v, sem).wait()
        pltpu.sync_copy(rows_v, out_hbm.at[pl.ds(base, b_per_w)])

    return gather_kernel


def _gather_rows(table, idx):
    return _make_sc_gather(idx.shape[0], table.shape[1])(table, idx)


def _knn_kernel(qx_ref, qb_ref, sxt_ref, sn_ref, bf_ref, col_ref):
    qx = qx_ref[...]
    dot = (qx[:, 0:1] * sxt_ref[0:1, :]
           + qx[:, 1:2] * sxt_ref[1:2, :]
           + qx[:, 2:3] * sxt_ref[2:3, :])
    d2 = sn_ref[...] - 2.0 * dot
    same = qb_ref[...] == bf_ref[...]
    d2 = jnp.where(same, d2, INF)
    col_iota = lax.broadcasted_iota(jnp.int32, (QTILE, NP), 1)
    for j in range(KNN):
        m = jnp.min(d2, axis=1, keepdims=True)
        hit = d2 == m
        idx = jnp.min(jnp.where(hit, col_iota, NP), axis=1, keepdims=True)
        col_ref[:, j : j + 1] = idx
        d2 = jnp.where(col_iota == idx, INF, d2)


def _knn(qx3, qb, sxt, sn, bf):
    grid = Q // QTILE
    return pl.pallas_call(
        _knn_kernel,
        grid=(grid,),
        in_specs=[
            pl.BlockSpec((QTILE, 3), lambda i: (i, 0)),
            pl.BlockSpec((QTILE, 1), lambda i: (i, 0)),
            pl.BlockSpec((3, NP), lambda i: (0, 0)),
            pl.BlockSpec((1, NP), lambda i: (0, 0)),
            pl.BlockSpec((1, NP), lambda i: (0, 0)),
        ],
        out_specs=pl.BlockSpec((QTILE, KNN), lambda i: (i, 0)),
        out_shape=jax.ShapeDtypeStruct((Q, KNN), jnp.int32),
    )(qx3, qb, sxt, sn, bf)


def _affinity_kernel(rows_ref, qx_ref, leaf_ref, w1_ref, b1_ref, w2_ref,
                     b2_ref, wl_ref, bl_ref, out_ref, matf_ref, lcat_ref,
                     rcat_ref):
    P = Q * KNN
    feat_j = rows_ref[:, :H]
    cls_in = rows_ref[:, H : H + NC]
    xyz_j = rows_ref[:, 141:144]
    qx = qx_ref[...]
    xyz_i = jnp.broadcast_to(
        qx.reshape(Q, 1, 3), (Q, KNN, 3)).reshape(P, 3)
    diff = xyz_i - xyz_j
    dn = jnp.sqrt(jnp.sum(diff * diff, axis=1, keepdims=True))

    h1 = (jnp.dot(xyz_i, w1_ref[0:3, :], preferred_element_type=jnp.float32)
          + jnp.dot(xyz_j, w1_ref[3:6, :], preferred_element_type=jnp.float32)
          + jnp.dot(diff, w1_ref[6:9, :], preferred_element_type=jnp.float32)
          + dn * w1_ref[9:10, :] + b1_ref[...])
    h1 = jnp.maximum(h1, 0.0)
    edge_w = jnp.dot(h1, w2_ref[...],
                     preferred_element_type=jnp.float32) + b2_ref[...]
    cls_w = jnp.dot(cls_in, wl_ref[...],
                    preferred_element_type=jnp.float32) + bl_ref[...]

    def _rownorm(x):
        return jnp.maximum(jnp.sqrt(jnp.sum(x * x, axis=1, keepdims=True)),
                           1e-8)

    leaf = leaf_ref[...]
    l1 = leaf[:, :H]
    l2 = leaf[:, H : 2 * H]
    l3 = leaf[:, 2 * H :]
    lcat_ref[:, :H] = l1 / (4.0 * _rownorm(l1))
    lcat_ref[:, H : 2 * H] = l2 / (4.0 * _rownorm(l2))
    lcat_ref[:, 2 * H :] = l3 / (2.0 * _rownorm(l3))
    rcat_ref[:, :H] = edge_w / _rownorm(edge_w)
    rcat_ref[:, H : 2 * H] = cls_w / _rownorm(cls_w)
    rcat_ref[:, 2 * H :] = feat_j / _rownorm(feat_j)

    GR = 256
    r_iota = lax.broadcasted_iota(jnp.int32, (GR, GR), 0)
    c_iota = lax.broadcasted_iota(jnp.int32, (GR, GR), 1)
    blockmask = (r_iota // L == c_iota // L).astype(jnp.float32)
    sel = (lax.broadcasted_iota(jnp.int32, (GR, L), 0) % L
           == lax.broadcasted_iota(jnp.int32, (GR, L), 1)).astype(jnp.float32)
    for g in range(Q // L):
        a = lcat_ref[g * GR : (g + 1) * GR, :]
        b = rcat_ref[g * GR : (g + 1) * GR, :]
        r = jnp.dot(a, b.T, preferred_element_type=jnp.float32)
        matf_ref[g * GR : (g + 1) * GR, :] = jnp.dot(
            r * blockmask, sel, preferred_element_type=jnp.float32,
            precision=lax.Precision.HIGHEST)

    mat3 = matf_ref[...].reshape(Q, L, KNN)
    eps = jnp.float32(1.0 / L)
    iota_k = lax.broadcasted_iota(jnp.int32, (Q, L, KNN), 2)
    iota_l = lax.broadcasted_iota(jnp.int32, (Q, L, KNN), 1)

    def body(carry):
        _, cnt, cost, ass = carry
        value = mat3 - cost
        m1 = jnp.max(value, axis=2, keepdims=True)
        i1 = jnp.min(jnp.where(value == m1, iota_k, KNN), axis=2,
                     keepdims=True)
        m2 = jnp.max(jnp.where(iota_k == i1, -INF, value), axis=2,
                     keepdims=True)
        bid = m1 - m2 + eps
        ass3 = ass.reshape(Q, L, 1)
        bids = jnp.where((iota_k == i1) & (ass3 == -1), bid, 0.0)
        hb = jnp.max(bids, axis=1, keepdims=True)
        have = hb > 0.0
        bidder = jnp.min(jnp.where(bids == hb, iota_l, L), axis=1,
                         keepdims=True)
        cost = cost + jnp.where(have, hb, 0.0)
        ha = jnp.max(jnp.where(
            (iota_k == jnp.clip(ass3, 0, KNN - 1)) & have, 1, 0), axis=2)
        ass = jnp.where((ass >= 0) & (ha > 0), -1, ass)
        newk = jnp.min(jnp.where((bidder == iota_l) & have, iota_k, KNN),
                       axis=2)
        ass = jnp.where(newk < KNN, newk, ass)
        return jnp.min(ass), cnt + 1, cost, ass

    def cond(carry):
        low, cnt, _, _ = carry
        return jnp.logical_and(low == -1, cnt < 1000)

    cost0 = jnp.zeros((Q, 1, KNN), jnp.float32)
    ass0 = jnp.full((Q, L), -1, jnp.int32)
    _, _, _, ass = lax.while_loop(cond, body, (jnp.int32(-1), jnp.int32(0),
                                               cost0, ass0))
    gm = jnp.clip(ass, 0, KNN - 1).reshape(Q, L, 1)
    matched = jnp.sum(jnp.where(iota_k == gm, mat3, 0.0), axis=2)
    out_ref[...] = jnp.sum(matched, axis=1, keepdims=True) / L


def _affinity(rows, qx3, leaf2d, W1, b1, W2, b2, Wl, bl):
    P = Q * KNN
    whole = lambda shape: pl.BlockSpec(shape, lambda: tuple(0 for _ in shape))
    return pl.pallas_call(
        _affinity_kernel,
        in_specs=[
            whole((P, TD)),
            whole((Q, 3)),
            whole((P, 3 * H)),
            whole((10, 64)),
            whole((1, 64)),
            whole((64, H)),
            whole((1, H)),
            whole((NC, H)),
            whole((1, H)),
        ],
        out_specs=whole((Q, 1)),
        out_shape=jax.ShapeDtypeStruct((Q, 1), jnp.float32),
        scratch_shapes=[
            pltpu.VMEM((P, KNN), jnp.float32),
            pltpu.VMEM((P, 3 * H), jnp.float32),
            pltpu.VMEM((P, 3 * H), jnp.float32),
        ],
    )(rows, qx3, leaf2d, W1, b1, W2, b2, Wl, bl)


def kernel(support_xyz, batch_index, filtered_index, gcnfeat, leaf_node_all,
           W1, b1, W2, b2, Wl, bl):
    batchf = batch_index.astype(jnp.float32)
    table = jnp.concatenate(
        [gcnfeat, support_xyz, batchf[:, None],
         jnp.zeros((N, TD - (H + NC) - 4), jnp.float32)], axis=1)

    qrows = _gather_rows(table, filtered_index.astype(jnp.int32))
    qx3 = qrows[:, 141:144]
    qb = qrows[:, 144:145]

    sxt = jnp.pad(support_xyz.T, ((0, 0), (0, NP - N)))
    sn = jnp.pad(jnp.sum(support_xyz * support_xyz, axis=1),
                 (0, NP - N))[None, :]
    bf = jnp.pad(batchf, (0, NP - N),
                 constant_values=-1.0)[None, :]

    col = _knn(qx3, qb, sxt, sn, bf)

    rows = _gather_rows(table, col.reshape(Q * KNN))
    leaf2d = leaf_node_all.reshape(Q * KNN, 3 * H)

    out = _affinity(rows, qx3, leaf2d, W1, b1[None, :], W2, b2[None, :],
                    Wl, bl[None, :])
    return out.reshape(Q)


# speedup vs baseline: 1.9426x; 1.9426x over previous
import functools

import jax
import jax.numpy as jnp
from jax import lax
from jax.experimental import pallas as pl
from jax.experimental.pallas import tpu as pltpu
from jax.experimental.pallas import tpu_sc as plsc

N = 50000
Q = 512
KNN = 16
H = 128
NC = 13
L = 16
NP = 50048
TD = 160
SC_CORES = 2
SC_SUBCORES = 16
NW = SC_CORES * SC_SUBCORES
QTILE = 128
INF = float("inf")


def _make_sc_gather(B, D):
    b_per_w = B // NW
    mesh = plsc.VectorSubcoreMesh(core_axis_name="c", subcore_axis_name="s")

    @functools.partial(
        pl.kernel,
        out_type=jax.ShapeDtypeStruct((B, D), jnp.float32),
        mesh=mesh,
        scratch_types=[
            pltpu.VMEM((b_per_w,), jnp.int32),
            pltpu.VMEM((b_per_w, D), jnp.float32),
            pltpu.SemaphoreType.DMA,
        ],
        compiler_params=pltpu.CompilerParams(use_tc_tiling_on_sc=False),
    )
    def gather_kernel(table_hbm, idx_hbm, out_hbm, idx_v, rows_v, sem):
        wid = lax.axis_index("s") * SC_CORES + lax.axis_index("c")
        base = wid * b_per_w
        pltpu.sync_copy(idx_hbm.at[pl.ds(base, b_per_w)], idx_v)
        pltpu.async_copy(table_hbm.at[idx_v], rows_v, sem).wait()
        pltpu.sync_copy(rows_v, out_hbm.at[pl.ds(base, b_per_w)])

    return gather_kernel


def _gather_rows(table, idx):
    return _make_sc_gather(idx.shape[0], table.shape[1])(table, idx)


CW = 2944
NCH = NP // CW


def _knn_kernel(qx_ref, nq_ref, qb_ref, sxt_ref, sn_ref, bf_ref, col_ref,
                d2_ref):
    qx = qx_ref[...]
    qb = qb_ref[...]
    nq = nq_ref[...]

    def fill(c, _):
        off = c * CW
        sx = sxt_ref[:, pl.ds(off, CW)]
        dot = (qx[:, 0:1] * sx[0:1, :] + qx[:, 1:2] * sx[1:2, :]
               + qx[:, 2:3] * sx[2:3, :])
        d2 = (nq + sn_ref[:, pl.ds(off, CW)]) - 2.0 * dot
        d2_ref[:, pl.ds(off, CW)] = jnp.where(
            qb == bf_ref[:, pl.ds(off, CW)], d2, INF)
        return 0

    lax.fori_loop(0, NCH, fill, 0)

    iota_c = lax.broadcasted_iota(jnp.int32, (QTILE, CW), 1)
    lane_k = lax.broadcasted_iota(jnp.int32, (QTILE, KNN), 1)

    def extract(j, carry):
        prev_idx, cols_acc = carry

        def scan(c, mcarry):
            m, idx = mcarry
            off = c * CW
            gi = iota_c + off
            ch = d2_ref[:, pl.ds(off, CW)]
            ch = jnp.where(gi == prev_idx, INF, ch)
            d2_ref[:, pl.ds(off, CW)] = ch
            cmin = jnp.min(ch, axis=1, keepdims=True)
            lidx = jnp.min(jnp.where(ch == cmin, gi, NP), axis=1,
                           keepdims=True)
            better = jnp.logical_or(cmin < m,
                                    jnp.logical_and(cmin == m, lidx < idx))
            return (jnp.where(better, cmin, m),
                    jnp.where(better, lidx, idx))

        m0 = jnp.full((QTILE, 1), INF, jnp.float32)
        i0 = jnp.full((QTILE, 1), NP, jnp.int32)
        _, idx = lax.fori_loop(0, NCH, scan, (m0, i0))
        cols_acc = jnp.where(lane_k == j,
                             jnp.broadcast_to(idx, (QTILE, KNN)), cols_acc)
        return idx, cols_acc

    prev0 = jnp.full((QTILE, 1), -1, jnp.int32)
    cols0 = jnp.zeros((QTILE, KNN), jnp.int32)
    _, cols = lax.fori_loop(0, KNN, extract, (prev0, cols0))
    col_ref[...] = cols


def _knn(qxr, nq, qb, sxt, sn, bf):
    grid = Q // QTILE
    return pl.pallas_call(
        _knn_kernel,
        grid=(grid,),
        in_specs=[
            pl.BlockSpec((QTILE, 3), lambda i: (i, 0)),
            pl.BlockSpec((QTILE, 1), lambda i: (i, 0)),
            pl.BlockSpec((QTILE, 1), lambda i: (i, 0)),
            pl.BlockSpec((3, NP), lambda i: (0, 0)),
            pl.BlockSpec((1, NP), lambda i: (0, 0)),
            pl.BlockSpec((1, NP), lambda i: (0, 0)),
        ],
        out_specs=pl.BlockSpec((QTILE, KNN), lambda i: (i, 0)),
        out_shape=jax.ShapeDtypeStruct((Q, KNN), jnp.int32),
        scratch_shapes=[pltpu.VMEM((QTILE, NP), jnp.float32)],
    )(qxr, nq, qb, sxt, sn, bf)


def _affinity_kernel(rows_ref, qx_ref, leaf_ref, w1_ref, b1_ref, w2_ref,
                     b2_ref, wl_ref, bl_ref, out_ref, matf_ref):
    GR = 256
    w1 = w1_ref[...]
    b1 = b1_ref[...]
    w2 = w2_ref[...]
    b2 = b2_ref[...]
    wl = wl_ref[...]
    bl = bl_ref[...]

    def _rownorm(x):
        return jnp.maximum(jnp.sqrt(jnp.sum(x * x, axis=1, keepdims=True)),
                           1e-8)

    rowq = lax.broadcasted_iota(jnp.int32, (GR, 1), 0) // L

    def _bf(x):
        return x.astype(jnp.bfloat16)

    def _numer(x, y):
        r = lax.dot_general(_bf(x), _bf(y), (((1,), (1,)), ((), ())),
                            preferred_element_type=jnp.float32)
        acc = jnp.zeros((GR, KNN), jnp.float32)
        for p in range(L):
            acc = jnp.where(rowq == p, r[:, p * KNN : (p + 1) * KNN], acc)
        return acc

    def _den(nx, ny):
        ny_r = jnp.broadcast_to(ny.reshape(L, 1, KNN),
                                (L, L, KNN)).reshape(GR, KNN)
        return nx * ny_r

    def group(g, _):
        r0 = g * GR
        rows = rows_ref[pl.ds(r0, GR), :]
        leaf = leaf_ref[pl.ds(r0, GR), :]
        feat_j = rows[:, :H]
        cls_in = rows[:, H : H + NC]
        xyz_j = rows[:, 141:144]
        qx = qx_ref[pl.ds(g * L, L), :]
        xyz_i = jnp.broadcast_to(
            qx.reshape(L, 1, 3), (L, KNN, 3)).reshape(GR, 3)
        diff = xyz_i - xyz_j
        dn = jnp.sqrt(jnp.sum(diff * diff, axis=1, keepdims=True))

        ew = jnp.concatenate([xyz_i, xyz_j, diff, dn], axis=1)
        h1 = jnp.maximum(
            jnp.dot(_bf(ew), _bf(w1), preferred_element_type=jnp.float32)
            + b1, 0.0)
        edge_w = jnp.dot(_bf(h1), _bf(w2),
                         preferred_element_type=jnp.float32) + b2
        cls_w = jnp.dot(_bf(cls_in), _bf(wl),
                        preferred_element_type=jnp.float32) + bl

        l1 = leaf[:, :H]
        l2 = leaf[:, H : 2 * H]
        l3 = leaf[:, 2 * H :]
        mat1 = _numer(l1, edge_w) / _den(_rownorm(l1), _rownorm(edge_w))
        mat2 = _numer(l2, cls_w) / _den(_rownorm(l2), _rownorm(cls_w))
        mat3 = _numer(l3, feat_j) / _den(_rownorm(l3), _rownorm(feat_j))
        matf_ref[pl.ds(r0, GR), :] = (mat1 + mat2) / 4.0 + mat3 / 2.0
        return 0

    lax.fori_loop(0, Q // L, group, 0)

    mat3 = matf_ref[...].reshape(Q, L, KNN)
    eps = jnp.float32(1.0 / L)
    iota_k = lax.broadcasted_iota(jnp.int32, (Q, L, KNN), 2)
    iota_l = lax.broadcasted_iota(jnp.int32, (Q, L, KNN), 1)

    def body(carry):
        _, cnt, cost, ass = carry
        value = mat3 - cost
        m1 = jnp.max(value, axis=2, keepdims=True)
        i1 = jnp.min(jnp.where(value == m1, iota_k, KNN), axis=2,
                     keepdims=True)
        m2 = jnp.max(jnp.where(iota_k == i1, -INF, value), axis=2,
                     keepdims=True)
        bid = m1 - m2 + eps
        ass3 = ass.reshape(Q, L, 1)
        bids = jnp.where((iota_k == i1) & (ass3 == -1), bid, 0.0)
        hb = jnp.max(bids, axis=1, keepdims=True)
        have = hb > 0.0
        bidder = jnp.min(jnp.where(bids == hb, iota_l, L), axis=1,
                         keepdims=True)
        cost = cost + jnp.where(have, hb, 0.0)
        ha = jnp.max(jnp.where(
            (iota_k == jnp.clip(ass3, 0, KNN - 1)) & have, 1, 0), axis=2)
        ass = jnp.where((ass >= 0) & (ha > 0), -1, ass)
        newk = jnp.min(jnp.where((bidder == iota_l) & have, iota_k, KNN),
                       axis=2)
        ass = jnp.where(newk < KNN, newk, ass)
        return jnp.min(ass), cnt + 1, cost, ass

    def cond(carry):
        low, cnt, _, _ = carry
        return jnp.logical_and(low == -1, cnt < 1000)

    cost0 = jnp.zeros((Q, 1, KNN), jnp.float32)
    ass0 = jnp.full((Q, L), -1, jnp.int32)
    _, _, _, ass = lax.while_loop(cond, body, (jnp.int32(-1), jnp.int32(0),
                                               cost0, ass0))
    gm = jnp.clip(ass, 0, KNN - 1).reshape(Q, L, 1)
    matched = jnp.sum(jnp.where(iota_k == gm, mat3, 0.0), axis=2)
    out_ref[...] = jnp.sum(matched, axis=1, keepdims=True) / L


def _affinity(rows, qx3, leaf2d, W1, b1, W2, b2, Wl, bl):
    P = Q * KNN
    whole = lambda shape: pl.BlockSpec(shape, lambda: tuple(0 for _ in shape))
    return pl.pallas_call(
        _affinity_kernel,
        in_specs=[
            whole((P, TD)),
            whole((Q, 3)),
            whole((P, 3 * H)),
            whole((10, 64)),
            whole((1, 64)),
            whole((64, H)),
            whole((1, H)),
            whole((NC, H)),
            whole((1, H)),
        ],
        out_specs=whole((Q, 1)),
        out_shape=jax.ShapeDtypeStruct((Q, 1), jnp.float32),
        scratch_shapes=[
            pltpu.VMEM((P, KNN), jnp.float32),
        ],
    )(rows, qx3, leaf2d, W1, b1, W2, b2, Wl, bl)


def kernel(support_xyz, batch_index, filtered_index, gcnfeat, leaf_node_all,
           W1, b1, W2, b2, Wl, bl):
    batchf = batch_index.astype(jnp.float32)
    table = jnp.concatenate(
        [gcnfeat, support_xyz, batchf[:, None],
         jnp.zeros((N, TD - (H + NC) - 4), jnp.float32)], axis=1)

    qrows = _gather_rows(table, filtered_index.astype(jnp.int32))
    qx3 = qrows[:, 141:144]
    qb = qrows[:, 144:145]

    qxr = qx3.astype(jnp.bfloat16).astype(jnp.float32)
    nq = jnp.sum(qx3 ** 2, axis=1)[:, None]
    sxt = jnp.pad(support_xyz.T.astype(jnp.bfloat16).astype(jnp.float32),
                  ((0, 0), (0, NP - N)))
    sn = jnp.pad(jnp.sum(support_xyz ** 2, axis=1),
                 (0, NP - N))[None, :]
    bf = jnp.pad(batchf, (0, NP - N),
                 constant_values=-1.0)[None, :]

    col = _knn(qxr, nq, qb, sxt, sn, bf)

    rows = _gather_rows(table, col.reshape(Q * KNN))
    leaf2d = leaf_node_all.reshape(Q * KNN, 3 * H)

    out = _affinity(rows, qx3, leaf2d, W1, b1[None, :], W2, b2[None, :],
                    Wl, bl[None, :])
    return out.reshape(Q)


# speedup vs baseline: 2.9575x; 1.5224x over previous
import functools

import jax
import jax.numpy as jnp
from jax import lax
from jax.experimental import pallas as pl
from jax.experimental.pallas import tpu as pltpu
from jax.experimental.pallas import tpu_sc as plsc

N = 50000
Q = 512
KNN = 16
H = 128
NC = 13
L = 16
NP = 50048
TD = 256
SC_CORES = 2
SC_SUBCORES = 16
NW = SC_CORES * SC_SUBCORES
QTILE = 128
INF = float("inf")


def _make_sc_gather(B, D):
    b_per_w = B // NW
    mesh = plsc.VectorSubcoreMesh(core_axis_name="c", subcore_axis_name="s")

    @functools.partial(
        pl.kernel,
        out_type=jax.ShapeDtypeStruct((B, D), jnp.float32),
        mesh=mesh,
        scratch_types=[
            pltpu.VMEM((b_per_w,), jnp.int32),
            pltpu.VMEM((b_per_w, D), jnp.float32),
            pltpu.SemaphoreType.DMA,
        ],
    )
    def gather_kernel(table_hbm, idx_hbm, out_hbm, idx_v, rows_v, sem):
        wid = lax.axis_index("s") * SC_CORES + lax.axis_index("c")
        base = wid * b_per_w
        pltpu.sync_copy(idx_hbm.at[pl.ds(base, b_per_w)], idx_v)
        pltpu.async_copy(table_hbm.at[idx_v], rows_v, sem).wait()
        pltpu.sync_copy(rows_v, out_hbm.at[pl.ds(base, b_per_w)])

    return gather_kernel


def _gather_rows(table, idx):
    return _make_sc_gather(idx.shape[0], table.shape[1])(table, idx)


CW = 2944
NCH = NP // CW


def _knn_kernel(tb_ref, qx_ref, nq_ref, qb_ref, sxt_ref, sn_ref, bf_ref,
                col_ref, d2_ref):
    qx = qx_ref[...]
    qb = qb_ref[...]
    nq = nq_ref[...]
    pid = pl.program_id(0)
    c0 = tb_ref[pid, 0]
    c1 = tb_ref[pid, 1]

    def fill(c, _):
        off = c * CW
        sx = sxt_ref[:, pl.ds(off, CW)]
        dot = (qx[:, 0:1] * sx[0:1, :] + qx[:, 1:2] * sx[1:2, :]
               + qx[:, 2:3] * sx[2:3, :])
        d2 = (nq + sn_ref[:, pl.ds(off, CW)]) - 2.0 * dot
        d2_ref[:, pl.ds(off, CW)] = jnp.where(
            qb == bf_ref[:, pl.ds(off, CW)], d2, INF)
        return 0

    lax.fori_loop(c0, c1, fill, 0)

    iota_c = lax.broadcasted_iota(jnp.int32, (QTILE, CW), 1)
    lane_k = lax.broadcasted_iota(jnp.int32, (QTILE, KNN), 1)

    def extract(j, carry):
        prev_idx, cols_acc = carry

        def scan(c, mcarry):
            m, idx = mcarry
            off = c * CW
            gi = iota_c + off
            ch = d2_ref[:, pl.ds(off, CW)]
            ch = jnp.where(gi == prev_idx, INF, ch)
            d2_ref[:, pl.ds(off, CW)] = ch
            cmin = jnp.min(ch, axis=1, keepdims=True)
            lidx = jnp.min(jnp.where(ch == cmin, gi, NP), axis=1,
                           keepdims=True)
            better = jnp.logical_or(cmin < m,
                                    jnp.logical_and(cmin == m, lidx < idx))
            return (jnp.where(better, cmin, m),
                    jnp.where(better, lidx, idx))

        m0 = jnp.full((QTILE, 1), INF, jnp.float32)
        i0 = jnp.full((QTILE, 1), NP, jnp.int32)
        _, idx = lax.fori_loop(c0, c1, scan, (m0, i0))
        cols_acc = jnp.where(lane_k == j,
                             jnp.broadcast_to(idx, (QTILE, KNN)), cols_acc)
        return idx, cols_acc

    prev0 = jnp.full((QTILE, 1), -1, jnp.int32)
    cols0 = jnp.zeros((QTILE, KNN), jnp.int32)
    _, cols = lax.fori_loop(0, KNN, extract, (prev0, cols0))
    col_ref[...] = cols


def _knn(tb, qxr, nq, qb, sxt, sn, bf):
    grid = Q // QTILE
    grid_spec = pltpu.PrefetchScalarGridSpec(
        num_scalar_prefetch=1,
        grid=(grid,),
        in_specs=[
            pl.BlockSpec((QTILE, 3), lambda i, tb: (i, 0)),
            pl.BlockSpec((QTILE, 1), lambda i, tb: (i, 0)),
            pl.BlockSpec((QTILE, 1), lambda i, tb: (i, 0)),
            pl.BlockSpec((3, NP), lambda i, tb: (0, 0)),
            pl.BlockSpec((1, NP), lambda i, tb: (0, 0)),
            pl.BlockSpec((1, NP), lambda i, tb: (0, 0)),
        ],
        out_specs=pl.BlockSpec((QTILE, KNN), lambda i, tb: (i, 0)),
        scratch_shapes=[pltpu.VMEM((QTILE, NP), jnp.float32)],
    )
    return pl.pallas_call(
        _knn_kernel,
        grid_spec=grid_spec,
        out_shape=jax.ShapeDtypeStruct((Q, KNN), jnp.int32),
    )(tb, qxr, nq, qb, sxt, sn, bf)


def _affinity_kernel(rows_ref, qx_ref, leaf_ref, w1_ref, b1_ref, w2_ref,
                     b2_ref, wl_ref, bl_ref, out_ref, matf_ref):
    GR = 256
    w1 = w1_ref[...]
    b1 = b1_ref[...]
    w2 = w2_ref[...]
    b2 = b2_ref[...]
    wl = wl_ref[...]
    bl = bl_ref[...]

    def _rownorm(x):
        return jnp.maximum(jnp.sqrt(jnp.sum(x * x, axis=1, keepdims=True)),
                           1e-8)

    rowq = lax.broadcasted_iota(jnp.int32, (GR, 1), 0) // L

    def _bf(x):
        return x.astype(jnp.bfloat16)

    def _numer(x, y):
        r = lax.dot_general(_bf(x), _bf(y), (((1,), (1,)), ((), ())),
                            preferred_element_type=jnp.float32)
        acc = jnp.zeros((GR, KNN), jnp.float32)
        for p in range(L):
            acc = jnp.where(rowq == p, r[:, p * KNN : (p + 1) * KNN], acc)
        return acc

    def _den(nx, ny):
        ny_r = jnp.broadcast_to(ny.reshape(L, 1, KNN),
                                (L, L, KNN)).reshape(GR, KNN)
        return nx * ny_r

    def group(g, _):
        r0 = g * GR
        rows = rows_ref[pl.ds(r0, GR), :]
        leaf = leaf_ref[pl.ds(r0, GR), :]
        feat_j = rows[:, :H]
        cls_in = rows[:, H : H + NC]
        xyz_j = rows[:, 141:144]
        qx = qx_ref[pl.ds(g * L, L), :]
        xyz_i = jnp.broadcast_to(
            qx.reshape(L, 1, 3), (L, KNN, 3)).reshape(GR, 3)
        diff = xyz_i - xyz_j
        dn = jnp.sqrt(jnp.sum(diff * diff, axis=1, keepdims=True))

        ew = jnp.concatenate([xyz_i, xyz_j, diff, dn], axis=1)
        h1 = jnp.maximum(
            jnp.dot(_bf(ew), _bf(w1), preferred_element_type=jnp.float32)
            + b1, 0.0)
        edge_w = jnp.dot(_bf(h1), _bf(w2),
                         preferred_element_type=jnp.float32) + b2
        cls_w = jnp.dot(_bf(cls_in), _bf(wl),
                        preferred_element_type=jnp.float32) + bl

        l1 = leaf[:, :H]
        l2 = leaf[:, H : 2 * H]
        l3 = leaf[:, 2 * H :]
        mat1 = _numer(l1, edge_w) / _den(_rownorm(l1), _rownorm(edge_w))
        mat2 = _numer(l2, cls_w) / _den(_rownorm(l2), _rownorm(cls_w))
        mat3 = _numer(l3, feat_j) / _den(_rownorm(l3), _rownorm(feat_j))
        matf_ref[pl.ds(r0, GR), :] = (mat1 + mat2) / 4.0 + mat3 / 2.0
        return 0

    lax.fori_loop(0, Q // L, group, 0)

    mat3 = matf_ref[...].reshape(Q, L, KNN)
    eps = jnp.float32(1.0 / L)
    iota_k = lax.broadcasted_iota(jnp.int32, (Q, L, KNN), 2)
    iota_l = lax.broadcasted_iota(jnp.int32, (Q, L, KNN), 1)

    def body(carry):
        _, cnt, cost, ass = carry
        value = mat3 - cost
        m1 = jnp.max(value, axis=2, keepdims=True)
        i1 = jnp.min(jnp.where(value == m1, iota_k, KNN), axis=2,
                     keepdims=True)
        m2 = jnp.max(jnp.where(iota_k == i1, -INF, value), axis=2,
                     keepdims=True)
        bid = m1 - m2 + eps
        ass3 = ass.reshape(Q, L, 1)
        bids = jnp.where((iota_k == i1) & (ass3 == -1), bid, 0.0)
        hb = jnp.max(bids, axis=1, keepdims=True)
        have = hb > 0.0
        bidder = jnp.min(jnp.where(bids == hb, iota_l, L), axis=1,
                         keepdims=True)
        cost = cost + jnp.where(have, hb, 0.0)
        ha = jnp.max(jnp.where(
            (iota_k == jnp.clip(ass3, 0, KNN - 1)) & have, 1, 0), axis=2)
        ass = jnp.where((ass >= 0) & (ha > 0), -1, ass)
        newk = jnp.min(jnp.where((bidder == iota_l) & have, iota_k, KNN),
                       axis=2)
        ass = jnp.where(newk < KNN, newk, ass)
        return jnp.min(ass), cnt + 1, cost, ass

    def cond(carry):
        low, cnt, _, _ = carry
        return jnp.logical_and(low == -1, cnt < 1000)

    cost0 = jnp.zeros((Q, 1, KNN), jnp.float32)
    ass0 = jnp.full((Q, L), -1, jnp.int32)
    _, _, _, ass = lax.while_loop(cond, body, (jnp.int32(-1), jnp.int32(0),
                                               cost0, ass0))
    gm = jnp.clip(ass, 0, KNN - 1).reshape(Q, L, 1)
    matched = jnp.sum(jnp.where(iota_k == gm, mat3, 0.0), axis=2)
    out_ref[...] = jnp.sum(matched, axis=1, keepdims=True) / L


def _affinity(rows, qx3, leaf2d, W1, b1, W2, b2, Wl, bl):
    P = Q * KNN
    whole = lambda shape: pl.BlockSpec(shape, lambda: tuple(0 for _ in shape))
    return pl.pallas_call(
        _affinity_kernel,
        in_specs=[
            whole((P, TD)),
            whole((Q, 3)),
            whole((P, 3 * H)),
            whole((10, 64)),
            whole((1, 64)),
            whole((64, H)),
            whole((1, H)),
            whole((NC, H)),
            whole((1, H)),
        ],
        out_specs=whole((Q, 1)),
        out_shape=jax.ShapeDtypeStruct((Q, 1), jnp.float32),
        scratch_shapes=[
            pltpu.VMEM((P, KNN), jnp.float32),
        ],
    )(rows, qx3, leaf2d, W1, b1, W2, b2, Wl, bl)


def kernel(support_xyz, batch_index, filtered_index, gcnfeat, leaf_node_all,
           W1, b1, W2, b2, Wl, bl):
    batchf = batch_index.astype(jnp.float32)
    table = jnp.concatenate(
        [gcnfeat, support_xyz, batchf[:, None],
         jnp.zeros((N, TD - (H + NC) - 4), jnp.float32)], axis=1)
    NBMAX = 8

    qrows = _gather_rows(table, filtered_index.astype(jnp.int32))
    qx3 = qrows[:, 141:144]
    qb = qrows[:, 144:145]

    qxr = qx3.astype(jnp.bfloat16).astype(jnp.float32)
    nq = jnp.sum(qx3 ** 2, axis=1)[:, None]
    sxt = jnp.pad(support_xyz.T.astype(jnp.bfloat16).astype(jnp.float32),
                  ((0, 0), (0, NP - N)))
    sn = jnp.pad(jnp.sum(support_xyz ** 2, axis=1),
                 (0, NP - N))[None, :]
    bf = jnp.pad(batchf, (0, NP - N),
                 constant_values=-1.0)[None, :]

    qbi = qb[:, 0].astype(jnp.int32)
    perm = jnp.argsort(qbi, stable=True)
    inv = jnp.argsort(perm, stable=True)
    qbs = qbi[perm]
    bids = jnp.arange(NBMAX, dtype=batch_index.dtype)
    starts = jnp.searchsorted(batch_index, bids, side="left")
    ends = jnp.searchsorted(batch_index, bids, side="right")
    lo = starts[qbs[0::QTILE]].astype(jnp.int32)
    hi = ends[qbs[QTILE - 1 :: QTILE]].astype(jnp.int32)
    tb = jnp.stack([lo // CW, (hi + CW - 1) // CW], axis=1)

    col_s = _knn(tb, qxr[perm], nq[perm], qb[perm], sxt, sn, bf)
    col = col_s[inv]

    rows = _gather_rows(table, col.reshape(Q * KNN))
    leaf2d = leaf_node_all.reshape(Q * KNN, 3 * H)

    out = _affinity(rows, qx3, leaf2d, W1, b1[None, :], W2, b2[None, :],
                    Wl, bl[None, :])
    return out.reshape(Q)


# speedup vs baseline: 3.5840x; 1.2118x over previous
import functools

import jax
import jax.numpy as jnp
from jax import lax
from jax.experimental import pallas as pl
from jax.experimental.pallas import tpu as pltpu
from jax.experimental.pallas import tpu_sc as plsc

N = 50000
Q = 512
KNN = 16
H = 128
NC = 13
L = 16
NP = 50048
TD = 256
SC_CORES = 2
SC_SUBCORES = 16
NW = SC_CORES * SC_SUBCORES
QTILE = 128
INF = float("inf")


def _make_sc_gather(B, D):
    b_per_w = B // NW
    mesh = plsc.VectorSubcoreMesh(core_axis_name="c", subcore_axis_name="s")

    @functools.partial(
        pl.kernel,
        out_type=jax.ShapeDtypeStruct((B, D), jnp.float32),
        mesh=mesh,
        scratch_types=[
            pltpu.VMEM((b_per_w,), jnp.int32),
            pltpu.VMEM((b_per_w, D), jnp.float32),
            pltpu.SemaphoreType.DMA,
        ],
    )
    def gather_kernel(table_hbm, idx_hbm, out_hbm, idx_v, rows_v, sem):
        wid = lax.axis_index("s") * SC_CORES + lax.axis_index("c")
        base = wid * b_per_w
        pltpu.sync_copy(idx_hbm.at[pl.ds(base, b_per_w)], idx_v)
        pltpu.async_copy(table_hbm.at[idx_v], rows_v, sem).wait()
        pltpu.sync_copy(rows_v, out_hbm.at[pl.ds(base, b_per_w)])

    return gather_kernel


def _gather_rows(table, idx):
    return _make_sc_gather(idx.shape[0], table.shape[1])(table, idx)


PBLK = 2000


def _pack_kernel(feat_ref, xyz_ref, bt_ref, out_ref):
    out_ref[:, 0 : H + NC] = feat_ref[...]
    out_ref[:, H + NC : H + NC + 3] = xyz_ref[...]
    out_ref[:, 144:145] = bt_ref[...]
    out_ref[:, 145:TD] = jnp.zeros((PBLK, TD - 145), jnp.float32)


def _pack_table(gcnfeat, support_xyz, batchf):
    return pl.pallas_call(
        _pack_kernel,
        grid=(N // PBLK,),
        in_specs=[
            pl.BlockSpec((PBLK, H + NC), lambda i: (i, 0)),
            pl.BlockSpec((PBLK, 3), lambda i: (i, 0)),
            pl.BlockSpec((PBLK, 1), lambda i: (i, 0)),
        ],
        out_specs=pl.BlockSpec((PBLK, TD), lambda i: (i, 0)),
        out_shape=jax.ShapeDtypeStruct((N, TD), jnp.float32),
    )(gcnfeat, support_xyz, batchf[:, None])


CW = 2944
NCH = NP // CW


def _knn_kernel(tb_ref, qx_ref, nq_ref, qb_ref, sxt_ref, sn_ref, bf_ref,
                col_ref, d2_ref):
    qx = qx_ref[...]
    qb = qb_ref[...]
    nq = nq_ref[...]
    pid = pl.program_id(0)
    c0 = tb_ref[pid, 0]
    c1 = tb_ref[pid, 1]

    def fill(c, _):
        off = c * CW
        sx = sxt_ref[:, pl.ds(off, CW)]
        dot = (qx[:, 0:1] * sx[0:1, :] + qx[:, 1:2] * sx[1:2, :]
               + qx[:, 2:3] * sx[2:3, :])
        d2 = (nq + sn_ref[:, pl.ds(off, CW)]) - 2.0 * dot
        d2_ref[:, pl.ds(off, CW)] = jnp.where(
            qb == bf_ref[:, pl.ds(off, CW)], d2, INF)
        return 0

    lax.fori_loop(c0, c1, fill, 0)

    iota_c = lax.broadcasted_iota(jnp.int32, (QTILE, CW), 1)
    lane_k = lax.broadcasted_iota(jnp.int32, (QTILE, KNN), 1)

    def extract(j, carry):
        prev_idx, cols_acc = carry

        def scan(c, mcarry):
            m, idx = mcarry
            off = c * CW
            gi = iota_c + off
            ch = d2_ref[:, pl.ds(off, CW)]
            ch = jnp.where(gi == prev_idx, INF, ch)
            d2_ref[:, pl.ds(off, CW)] = ch
            cmin = jnp.min(ch, axis=1, keepdims=True)
            lidx = jnp.min(jnp.where(ch == cmin, gi, NP), axis=1,
                           keepdims=True)
            better = jnp.logical_or(cmin < m,
                                    jnp.logical_and(cmin == m, lidx < idx))
            return (jnp.where(better, cmin, m),
                    jnp.where(better, lidx, idx))

        m0 = jnp.full((QTILE, 1), INF, jnp.float32)
        i0 = jnp.full((QTILE, 1), NP, jnp.int32)
        _, idx = lax.fori_loop(c0, c1, scan, (m0, i0))
        cols_acc = jnp.where(lane_k == j,
                             jnp.broadcast_to(idx, (QTILE, KNN)), cols_acc)
        return idx, cols_acc

    prev0 = jnp.full((QTILE, 1), -1, jnp.int32)
    cols0 = jnp.zeros((QTILE, KNN), jnp.int32)
    _, cols = lax.fori_loop(0, KNN, extract, (prev0, cols0))
    col_ref[...] = cols


def _knn(tb, qxr, nq, qb, sxt, sn, bf):
    grid = Q // QTILE
    grid_spec = pltpu.PrefetchScalarGridSpec(
        num_scalar_prefetch=1,
        grid=(grid,),
        in_specs=[
            pl.BlockSpec((QTILE, 3), lambda i, tb: (i, 0)),
            pl.BlockSpec((QTILE, 1), lambda i, tb: (i, 0)),
            pl.BlockSpec((QTILE, 1), lambda i, tb: (i, 0)),
            pl.BlockSpec((3, NP), lambda i, tb: (0, 0)),
            pl.BlockSpec((1, NP), lambda i, tb: (0, 0)),
            pl.BlockSpec((1, NP), lambda i, tb: (0, 0)),
        ],
        out_specs=pl.BlockSpec((QTILE, KNN), lambda i, tb: (i, 0)),
        scratch_shapes=[pltpu.VMEM((QTILE, NP), jnp.float32)],
    )
    return pl.pallas_call(
        _knn_kernel,
        grid_spec=grid_spec,
        out_shape=jax.ShapeDtypeStruct((Q, KNN), jnp.int32),
    )(tb, qxr, nq, qb, sxt, sn, bf)


def _affinity_kernel(rows_ref, qx_ref, leaf_ref, w1_ref, b1_ref, w2_ref,
                     b2_ref, wl_ref, bl_ref, out_ref, matf_ref):
    GR = 256
    w1 = w1_ref[...]
    b1 = b1_ref[...]
    w2 = w2_ref[...]
    b2 = b2_ref[...]
    wl = wl_ref[...]
    bl = bl_ref[...]

    def _rownorm(x):
        return jnp.maximum(jnp.sqrt(jnp.sum(x * x, axis=1, keepdims=True)),
                           1e-8)

    rowq = lax.broadcasted_iota(jnp.int32, (GR, 1), 0) // L

    def _bf(x):
        return x.astype(jnp.bfloat16)

    def _numer(x, y):
        r = lax.dot_general(_bf(x), _bf(y), (((1,), (1,)), ((), ())),
                            preferred_element_type=jnp.float32)
        acc = jnp.zeros((GR, KNN), jnp.float32)
        for p in range(L):
            acc = jnp.where(rowq == p, r[:, p * KNN : (p + 1) * KNN], acc)
        return acc

    def _den(nx, ny):
        ny_r = jnp.broadcast_to(ny.reshape(L, 1, KNN),
                                (L, L, KNN)).reshape(GR, KNN)
        return nx * ny_r

    def group(g, _):
        r0 = g * GR
        rows = rows_ref[pl.ds(r0, GR), :]
        leaf = leaf_ref[pl.ds(r0, GR), :]
        feat_j = rows[:, :H]
        cls_in = rows[:, H : H + NC]
        xyz_j = rows[:, 141:144]
        qx = qx_ref[pl.ds(g * L, L), :]
        xyz_i = jnp.broadcast_to(
            qx.reshape(L, 1, 3), (L, KNN, 3)).reshape(GR, 3)
        diff = xyz_i - xyz_j
        dn = jnp.sqrt(jnp.sum(diff * diff, axis=1, keepdims=True))

        ew = jnp.concatenate([xyz_i, xyz_j, diff, dn], axis=1)
        h1 = jnp.maximum(
            jnp.dot(_bf(ew), _bf(w1), preferred_element_type=jnp.float32)
            + b1, 0.0)
        edge_w = jnp.dot(_bf(h1), _bf(w2),
                         preferred_element_type=jnp.float32) + b2
        cls_w = jnp.dot(_bf(cls_in), _bf(wl),
                        preferred_element_type=jnp.float32) + bl

        l1 = leaf[:, :H]
        l2 = leaf[:, H : 2 * H]
        l3 = leaf[:, 2 * H :]
        mat1 = _numer(l1, edge_w) / _den(_rownorm(l1), _rownorm(edge_w))
        mat2 = _numer(l2, cls_w) / _den(_rownorm(l2), _rownorm(cls_w))
        mat3 = _numer(l3, feat_j) / _den(_rownorm(l3), _rownorm(feat_j))
        matf_ref[pl.ds(r0, GR), :] = (mat1 + mat2) / 4.0 + mat3 / 2.0
        return 0

    lax.fori_loop(0, Q // L, group, 0)

    mat3 = matf_ref[...].reshape(Q, L, KNN)
    eps = jnp.float32(1.0 / L)
    iota_k = lax.broadcasted_iota(jnp.int32, (Q, L, KNN), 2)
    iota_l = lax.broadcasted_iota(jnp.int32, (Q, L, KNN), 1)

    def body(carry):
        _, cnt, cost, ass = carry
        value = mat3 - cost
        m1 = jnp.max(value, axis=2, keepdims=True)
        i1 = jnp.min(jnp.where(value == m1, iota_k, KNN), axis=2,
                     keepdims=True)
        m2 = jnp.max(jnp.where(iota_k == i1, -INF, value), axis=2,
                     keepdims=True)
        bid = m1 - m2 + eps
        ass3 = ass.reshape(Q, L, 1)
        bids = jnp.where((iota_k == i1) & (ass3 == -1), bid, 0.0)
        hb = jnp.max(bids, axis=1, keepdims=True)
        have = hb > 0.0
        bidder = jnp.min(jnp.where(bids == hb, iota_l, L), axis=1,
                         keepdims=True)
        cost = cost + jnp.where(have, hb, 0.0)
        ha = jnp.max(jnp.where(
            (iota_k == jnp.clip(ass3, 0, KNN - 1)) & have, 1, 0), axis=2)
        ass = jnp.where((ass >= 0) & (ha > 0), -1, ass)
        newk = jnp.min(jnp.where((bidder == iota_l) & have, iota_k, KNN),
                       axis=2)
        ass = jnp.where(newk < KNN, newk, ass)
        return jnp.min(ass), cnt + 1, cost, ass

    def cond(carry):
        low, cnt, _, _ = carry
        return jnp.logical_and(low == -1, cnt < 1000)

    cost0 = jnp.zeros((Q, 1, KNN), jnp.float32)
    ass0 = jnp.full((Q, L), -1, jnp.int32)
    _, _, _, ass = lax.while_loop(cond, body, (jnp.int32(-1), jnp.int32(0),
                                               cost0, ass0))
    gm = jnp.clip(ass, 0, KNN - 1).reshape(Q, L, 1)
    matched = jnp.sum(jnp.where(iota_k == gm, mat3, 0.0), axis=2)
    out_ref[...] = jnp.sum(matched, axis=1, keepdims=True) / L


def _affinity(rows, qx3, leaf2d, W1, b1, W2, b2, Wl, bl):
    P = Q * KNN
    whole = lambda shape: pl.BlockSpec(shape, lambda: tuple(0 for _ in shape))
    return pl.pallas_call(
        _affinity_kernel,
        in_specs=[
            whole((P, TD)),
            whole((Q, 3)),
            whole((P, 3 * H)),
            whole((10, 64)),
            whole((1, 64)),
            whole((64, H)),
            whole((1, H)),
            whole((NC, H)),
            whole((1, H)),
        ],
        out_specs=whole((Q, 1)),
        out_shape=jax.ShapeDtypeStruct((Q, 1), jnp.float32),
        scratch_shapes=[
            pltpu.VMEM((P, KNN), jnp.float32),
        ],
    )(rows, qx3, leaf2d, W1, b1, W2, b2, Wl, bl)


def kernel(support_xyz, batch_index, filtered_index, gcnfeat, leaf_node_all,
           W1, b1, W2, b2, Wl, bl):
    batchf = batch_index.astype(jnp.float32)
    table = _pack_table(gcnfeat, support_xyz, batchf)
    NBMAX = 8

    qrows = _gather_rows(table, filtered_index.astype(jnp.int32))
    qx3 = qrows[:, 141:144]
    qb = qrows[:, 144:145]

    qxr = qx3.astype(jnp.bfloat16).astype(jnp.float32)
    nq = jnp.sum(qx3 ** 2, axis=1)[:, None]
    sxt = jnp.pad(support_xyz.T.astype(jnp.bfloat16).astype(jnp.float32),
                  ((0, 0), (0, NP - N)))
    sn = jnp.pad(jnp.sum(support_xyz ** 2, axis=1),
                 (0, NP - N))[None, :]
    bf = jnp.pad(batchf, (0, NP - N),
                 constant_values=-1.0)[None, :]

    qbi = qb[:, 0].astype(jnp.int32)
    perm = jnp.argsort(qbi, stable=True)
    inv = jnp.argsort(perm, stable=True)
    qbs = qbi[perm]
    bids = jnp.arange(NBMAX, dtype=batch_index.dtype)
    starts = jnp.searchsorted(batch_index, bids, side="left")
    ends = jnp.searchsorted(batch_index, bids, side="right")
    lo = starts[qbs[0::QTILE]].astype(jnp.int32)
    hi = ends[qbs[QTILE - 1 :: QTILE]].astype(jnp.int32)
    tb = jnp.stack([lo // CW, (hi + CW - 1) // CW], axis=1)

    col_s = _knn(tb, qxr[perm], nq[perm], qb[perm], sxt, sn, bf)
    col = col_s[inv]

    rows = _gather_rows(table, col.reshape(Q * KNN))
    leaf2d = leaf_node_all.reshape(Q * KNN, 3 * H)

    out = _affinity(rows, qx3, leaf2d, W1, b1[None, :], W2, b2[None, :],
                    Wl, bl[None, :])
    return out.reshape(Q)
